# Initial kernel scaffold; baseline (speedup 1.0000x reference)
import jax
import jax.numpy as jnp
from jax.experimental import pallas as pl


def kernel(x, relative_attention_bias_table):
    raise NotImplementedError("write your pallas kernel here")


# speedup vs baseline: 9.5658x; 9.5658x over previous
import functools

import jax
import jax.numpy as jnp
from jax import lax
from jax.experimental import pallas as pl
from jax.experimental.pallas import tpu as pltpu
from jax.experimental.pallas import tpu_sc as plsc

_NUM_HEADS = 16
_MAX_DISTANCE = 128


def kernel(x, relative_attention_bias_table):
    q_len = x.shape[1]
    n_table, n_heads = relative_attention_bias_table.shape
    max_rel = _MAX_DISTANCE - 1

    info = plsc.get_sparse_core_info()
    nc, ns = info.num_cores, info.num_subcores
    nw = nc * ns
    rows_per_w = q_len // nw
    w_rows = 2 * q_len
    rows_per_tile = w_rows // ns

    table_flat = relative_attention_bias_table.reshape(-1)

    mesh = plsc.VectorSubcoreMesh(core_axis_name="c", subcore_axis_name="s")

    @functools.partial(
        pl.kernel,
        mesh=mesh,
        out_type=jax.ShapeDtypeStruct((q_len * q_len * n_heads,), jnp.float32),
        scratch_types=[
            pltpu.VMEM((n_table * n_heads,), jnp.float32),
            pltpu.VMEM((rows_per_tile * n_heads,), jnp.float32),
            pltpu.VMEM_SHARED((w_rows * n_heads,), jnp.float32),
            pltpu.SemaphoreType.DMA,
        ],
        compiler_params=pltpu.CompilerParams(use_tc_tiling_on_sc=False),
    )
    def sc_kernel(table_hbm, out_hbm, table_v, wchunk_v, w_sh, sem):
        cid = lax.axis_index("c")
        sid = lax.axis_index("s")
        wid = sid * nc + cid

        pltpu.sync_copy(table_hbm, table_v)

        u_base = sid * rows_per_tile

        def build_row(r, carry):
            u = u_base + r
            rel = (q_len - 1) - u
            idx = jnp.clip(rel, -max_rel, max_rel) + max_rel
            wchunk_v[pl.ds(r * n_heads, n_heads)] = table_v[
                pl.ds(idx * n_heads, n_heads)
            ]
            return carry

        lax.fori_loop(0, rows_per_tile, build_row, 0)

        pltpu.sync_copy(wchunk_v, w_sh.at[pl.ds(u_base * n_heads, rows_per_tile * n_heads)])
        plsc.subcore_barrier()

        i0 = wid * rows_per_w
        row_elems = q_len * n_heads

        def row_group(g, carry):
            cps = []
            for b in range(8):
                i = i0 + g * 8 + b
                off = ((q_len - 1) - i) * n_heads
                cps.append(
                    pltpu.async_copy(
                        w_sh.at[pl.ds(off, row_elems)],
                        out_hbm.at[pl.ds(i * row_elems, row_elems)],
                        sem,
                    )
                )
            for cp in cps:
                cp.wait()
            return carry

        lax.fori_loop(0, rows_per_w // 8, row_group, 0)

    out_flat = sc_kernel(table_flat)
    return out_flat.reshape(q_len, q_len, n_heads)


# speedup vs baseline: 102.5599x; 10.7215x over previous
import functools

import jax
import jax.numpy as jnp
from jax import lax
from jax.experimental import pallas as pl
from jax.experimental.pallas import tpu as pltpu
from jax.experimental.pallas import tpu_sc as plsc

_NUM_HEADS = 16
_MAX_DISTANCE = 128


def _build_wt_sc(table_flat, q_len, n_heads):
    max_rel = _MAX_DISTANCE - 1
    info = plsc.get_sparse_core_info()
    nc, ns = info.num_cores, info.num_subcores
    nw = nc * ns
    w_cols = 2 * q_len
    halves = nw // n_heads
    cols_per_w = w_cols // halves

    mesh = plsc.VectorSubcoreMesh(core_axis_name="c", subcore_axis_name="s")

    @functools.partial(
        pl.kernel,
        mesh=mesh,
        out_type=jax.ShapeDtypeStruct((n_heads, w_cols), jnp.float32),
        scratch_types=[
            pltpu.VMEM((table_flat.shape[0],), jnp.float32),
            pltpu.VMEM((cols_per_w,), jnp.float32),
            pltpu.SemaphoreType.DMA,
        ],
        compiler_params=pltpu.CompilerParams(
            use_tc_tiling_on_sc=False, needs_layout_passes=False
        ),
    )
    def sc_kernel(table_hbm, wt_hbm, table_v, row_v, sem):
        wid = lax.axis_index("s") * nc + lax.axis_index("c")
        h = wid // halves
        half = wid % halves

        pltpu.sync_copy(table_hbm, table_v)

        lane = lax.iota(jnp.int32, 16)
        u_base = half * cols_per_w

        def build(k, carry):
            u = u_base + k * 16 + lane
            rel = (q_len - 1) - u
            idx = jnp.clip(rel, -max_rel, max_rel) + max_rel
            vals = plsc.load_gather(table_v, [idx * n_heads + h])
            row_v[pl.ds(k * 16, 16)] = vals
            return carry

        lax.fori_loop(0, cols_per_w // 16, build, 0)

        pltpu.sync_copy(row_v, wt_hbm.at[h, pl.ds(u_base, cols_per_w)])

    return sc_kernel(table_flat)


def kernel(x, relative_attention_bias_table):
    q_len = x.shape[1]
    n_table, n_heads = relative_attention_bias_table.shape

    wt = _build_wt_sc(relative_attention_bias_table.reshape(-1), q_len, n_heads)

    bq = 16

    def tc_body(wt_ref, out_ref):
        ib = pl.program_id(0)
        w = wt_ref[...]
        off0 = (q_len - 1) - ib * bq
        rolled = pltpu.roll(w, -off0, axis=1)
        for r in range(bq):
            shifted = pltpu.roll(rolled, r, axis=1) if r else rolled
            out_ref[r] = shifted[:, :q_len]

    out_t = pl.pallas_call(
        tc_body,
        grid=(q_len // bq,),
        in_specs=[pl.BlockSpec((n_heads, 2 * q_len), lambda i: (0, 0))],
        out_specs=pl.BlockSpec((bq, n_heads, q_len), lambda i: (i, 0, 0)),
        out_shape=jax.ShapeDtypeStruct((q_len, n_heads, q_len), jnp.float32),
    )(wt)

    return jnp.transpose(out_t, (0, 2, 1))


# speedup vs baseline: 124.1365x; 1.2104x over previous
import functools

import jax
import jax.numpy as jnp
from jax import lax
from jax.experimental import pallas as pl
from jax.experimental.pallas import tpu as pltpu
from jax.experimental.pallas import tpu_sc as plsc

_NUM_HEADS = 16
_MAX_DISTANCE = 128


def _build_wt_sc(table_flat, q_len, n_heads):
    max_rel = _MAX_DISTANCE - 1
    info = plsc.get_sparse_core_info()
    nc, ns = info.num_cores, info.num_subcores
    nw = nc * ns
    w_cols = 2 * q_len
    halves = nw // n_heads
    cols_per_w = w_cols // halves

    mesh = plsc.VectorSubcoreMesh(core_axis_name="c", subcore_axis_name="s")

    @functools.partial(
        pl.kernel,
        mesh=mesh,
        out_type=jax.ShapeDtypeStruct((n_heads, w_cols), jnp.float32),
        scratch_types=[
            pltpu.VMEM((table_flat.shape[0],), jnp.float32),
            pltpu.VMEM((cols_per_w,), jnp.float32),
            pltpu.SemaphoreType.DMA,
        ],
        compiler_params=pltpu.CompilerParams(
            use_tc_tiling_on_sc=False, needs_layout_passes=False
        ),
    )
    def sc_kernel(table_hbm, wt_hbm, table_v, row_v, sem):
        wid = lax.axis_index("s") * nc + lax.axis_index("c")
        h = wid // halves
        half = wid % halves

        pltpu.sync_copy(table_hbm, table_v)

        lane = lax.iota(jnp.int32, 16)
        u_base = half * cols_per_w

        def build(k, carry):
            u = u_base + k * 16 + lane
            rel = (q_len - 1) - u
            idx = jnp.clip(rel, -max_rel, max_rel) + max_rel
            vals = plsc.load_gather(table_v, [idx * n_heads + h])
            row_v[pl.ds(k * 16, 16)] = vals
            return carry

        lax.fori_loop(0, cols_per_w // 16, build, 0)

        pltpu.sync_copy(row_v, wt_hbm.at[h, pl.ds(u_base, cols_per_w)])

    return sc_kernel(table_flat)


def kernel(x, relative_attention_bias_table):
    q_len = x.shape[1]
    n_table, n_heads = relative_attention_bias_table.shape

    wt = _build_wt_sc(relative_attention_bias_table.reshape(-1), q_len, n_heads)

    bq = 32

    def tc_body(wt_ref, out_ref):
        ib = pl.program_id(0)
        w = wt_ref[...]
        off0 = (q_len - 1) - ib * bq
        rolled = pltpu.roll(w, -off0, axis=1)
        for r in range(bq):
            shifted = pltpu.roll(rolled, r, axis=1) if r else rolled
            out_ref[r] = shifted[:, :q_len]

    out_t = pl.pallas_call(
        tc_body,
        grid=(q_len // bq,),
        in_specs=[pl.BlockSpec((n_heads, 2 * q_len), lambda i: (0, 0))],
        out_specs=pl.BlockSpec((bq, n_heads, q_len), lambda i: (i, 0, 0)),
        out_shape=jax.ShapeDtypeStruct((q_len, n_heads, q_len), jnp.float32),
    )(wt)

    return jnp.transpose(out_t, (0, 2, 1))


# speedup vs baseline: 137.5989x; 1.1084x over previous
import functools

import jax
import jax.numpy as jnp
from jax import lax
from jax.experimental import pallas as pl
from jax.experimental.pallas import tpu as pltpu
from jax.experimental.pallas import tpu_sc as plsc

_NUM_HEADS = 16
_MAX_DISTANCE = 128


def _build_wt_sc(table_flat, q_len, n_heads):
    max_rel = _MAX_DISTANCE - 1
    info = plsc.get_sparse_core_info()
    nc, ns = info.num_cores, info.num_subcores
    nw = nc * ns
    w_cols = 2 * q_len
    halves = nw // n_heads
    cols_per_w = w_cols // halves

    mesh = plsc.VectorSubcoreMesh(core_axis_name="c", subcore_axis_name="s")

    @functools.partial(
        pl.kernel,
        mesh=mesh,
        out_type=jax.ShapeDtypeStruct((n_heads, w_cols), jnp.float32),
        scratch_types=[
            pltpu.VMEM((table_flat.shape[0],), jnp.float32),
            pltpu.VMEM((cols_per_w,), jnp.float32),
            pltpu.SemaphoreType.DMA,
        ],
        compiler_params=pltpu.CompilerParams(
            use_tc_tiling_on_sc=False, needs_layout_passes=False
        ),
    )
    def sc_kernel(table_hbm, wt_hbm, table_v, row_v, sem):
        wid = lax.axis_index("s") * nc + lax.axis_index("c")
        h = wid // halves
        half = wid % halves

        pltpu.sync_copy(table_hbm, table_v)

        lane = lax.iota(jnp.int32, 16)
        u_base = half * cols_per_w

        def build(k, carry):
            u = u_base + k * 16 + lane
            rel = (q_len - 1) - u
            idx = jnp.clip(rel, -max_rel, max_rel) + max_rel
            vals = plsc.load_gather(table_v, [idx * n_heads + h])
            row_v[pl.ds(k * 16, 16)] = vals
            return carry

        lax.fori_loop(0, cols_per_w // 16, build, 0)

        pltpu.sync_copy(row_v, wt_hbm.at[h, pl.ds(u_base, cols_per_w)])

    return sc_kernel(table_flat)


def kernel(x, relative_attention_bias_table):
    q_len = x.shape[1]
    n_table, n_heads = relative_attention_bias_table.shape

    wt = _build_wt_sc(relative_attention_bias_table.reshape(-1), q_len, n_heads)

    bq = 64

    def tc_body(wt_ref, out_ref):
        ib = pl.program_id(0)
        w = wt_ref[...]
        off0 = (q_len - 1) - ib * bq
        rolled = pltpu.roll(w, -off0, axis=1)
        for r in range(bq):
            shifted = pltpu.roll(rolled, r, axis=1) if r else rolled
            out_ref[r] = shifted[:, :q_len]

    out_t = pl.pallas_call(
        tc_body,
        grid=(q_len // bq,),
        in_specs=[pl.BlockSpec((n_heads, 2 * q_len), lambda i: (0, 0))],
        out_specs=pl.BlockSpec((bq, n_heads, q_len), lambda i: (i, 0, 0)),
        out_shape=jax.ShapeDtypeStruct((q_len, n_heads, q_len), jnp.float32),
    )(wt)

    return jnp.transpose(out_t, (0, 2, 1))


# speedup vs baseline: 141.7179x; 1.0299x over previous
import functools

import jax
import jax.numpy as jnp
from jax import lax
from jax.experimental import pallas as pl
from jax.experimental.pallas import tpu as pltpu
from jax.experimental.pallas import tpu_sc as plsc

_NUM_HEADS = 16
_MAX_DISTANCE = 128


def _build_wt_sc(table_flat, q_len, n_heads):
    max_rel = _MAX_DISTANCE - 1
    info = plsc.get_sparse_core_info()
    nc, ns = info.num_cores, info.num_subcores
    nw = nc * ns
    w_cols = 2 * q_len
    halves = nw // n_heads
    cols_per_w = w_cols // halves

    mesh = plsc.VectorSubcoreMesh(core_axis_name="c", subcore_axis_name="s")

    @functools.partial(
        pl.kernel,
        mesh=mesh,
        out_type=jax.ShapeDtypeStruct((n_heads, w_cols), jnp.float32),
        scratch_types=[
            pltpu.VMEM((table_flat.shape[0],), jnp.float32),
            pltpu.VMEM((cols_per_w,), jnp.float32),
            pltpu.SemaphoreType.DMA,
        ],
        compiler_params=pltpu.CompilerParams(
            use_tc_tiling_on_sc=False, needs_layout_passes=False
        ),
    )
    def sc_kernel(table_hbm, wt_hbm, table_v, row_v, sem):
        wid = lax.axis_index("s") * nc + lax.axis_index("c")
        h = wid // halves
        half = wid % halves

        pltpu.sync_copy(table_hbm, table_v)

        lane = lax.iota(jnp.int32, 16)
        u_base = half * cols_per_w

        def build(k, carry):
            u = u_base + k * 16 + lane
            rel = (q_len - 1) - u
            idx = jnp.clip(rel, -max_rel, max_rel) + max_rel
            vals = plsc.load_gather(table_v, [idx * n_heads + h])
            row_v[pl.ds(k * 16, 16)] = vals
            return carry

        lax.fori_loop(0, cols_per_w // 16, build, 0)

        pltpu.sync_copy(row_v, wt_hbm.at[h, pl.ds(u_base, cols_per_w)])

    return sc_kernel(table_flat)


def kernel(x, relative_attention_bias_table):
    q_len = x.shape[1]
    n_table, n_heads = relative_attention_bias_table.shape

    wt = _build_wt_sc(relative_attention_bias_table.reshape(-1), q_len, n_heads)

    bq = 128

    def tc_body(wt_ref, out_ref):
        ib = pl.program_id(0)
        w = wt_ref[...]
        off0 = (q_len - 1) - ib * bq
        rolled = pltpu.roll(w, -off0, axis=1)
        for r in range(bq):
            shifted = pltpu.roll(rolled, r, axis=1) if r else rolled
            out_ref[r] = shifted[:, :q_len]

    out_t = pl.pallas_call(
        tc_body,
        grid=(q_len // bq,),
        in_specs=[pl.BlockSpec((n_heads, 2 * q_len), lambda i: (0, 0))],
        out_specs=pl.BlockSpec((bq, n_heads, q_len), lambda i: (i, 0, 0)),
        out_shape=jax.ShapeDtypeStruct((q_len, n_heads, q_len), jnp.float32),
    )(wt)

    return jnp.transpose(out_t, (0, 2, 1))
